# Initial kernel scaffold; baseline (speedup 1.0000x reference)
import functools

import jax
import jax.numpy as jnp
from jax import lax
from jax.experimental import pallas as pl
from jax.experimental.pallas import tpu as pltpu
from jax.experimental.pallas import tpu_sc as plsc

NUM_CODES = 1024
EMB_DIM = 256
NUM_HEADS = 4
HEAD_DIM = EMB_DIM // NUM_HEADS
COMMITMENT_COST = 0.1

BATCH = 65536
ROWS_PER_BLOCK = 512

SC_CORES = 2
SC_SUBCORES = 16
SC_WORKERS = SC_CORES * SC_SUBCORES
TOTAL_ROWS = BATCH * NUM_HEADS
ROWS_PER_WORKER = TOTAL_ROWS // SC_WORKERS
SC_CHUNK = 512


def _tc_body(z_ref, zn_ref, wt_ref, wn_ref,
             idx_ref, gidx_ref, loss_ref, perp_ref,
             counts, loss_acc):
    step = pl.program_id(0)
    nsteps = pl.num_programs(0)

    @pl.when(step == 0)
    def _init():
        counts[...] = jnp.zeros_like(counts)
        loss_acc[0] = jnp.float32(0.0)

    idx_cols = []
    total = jnp.float32(0.0)
    for h in range(NUM_HEADS):
        zh = z_ref[:, h * HEAD_DIM:(h + 1) * HEAD_DIM]
        w = wt_ref[h]
        mm = jnp.dot(zh, w, preferred_element_type=jnp.float32)
        zn = zn_ref[:, h:h + 1]
        wn = wn_ref[h:h + 1, :]
        sq = (zn + wn) - 2.0 * mm
        dist = jnp.sqrt(jnp.maximum(sq, 0.0))
        m = jnp.min(dist, axis=1, keepdims=True)
        iota = lax.broadcasted_iota(jnp.int32, dist.shape, 1)
        idxh = jnp.min(jnp.where(dist == m, iota, jnp.int32(NUM_CODES)), axis=1)
        idx_cols.append(idxh)
        onehot = (iota == idxh[:, None]).astype(jnp.float32)
        counts[h:h + 1, :] = counts[h:h + 1, :] + jnp.sum(
            onehot, axis=0, keepdims=True)
        total = total + jnp.sum(m[:, 0] * m[:, 0])

    idx_blk = jnp.stack(idx_cols, axis=1)
    idx_ref[...] = idx_blk
    offs = jnp.arange(NUM_HEADS, dtype=jnp.int32) * NUM_CODES
    gidx_ref[...] = idx_blk + offs[None, :]
    loss_acc[0] = loss_acc[0] + total

    @pl.when(step == nsteps - 1)
    def _fin():
        avg = counts[...] * jnp.float32(1.0 / BATCH)
        ent = -jnp.sum(avg * jnp.log(avg + 1e-10), axis=1, keepdims=True)
        perp_ref[0] = jnp.mean(jnp.exp(ent))
        loss_ref[0] = loss_acc[0] * jnp.float32(1.0 / (BATCH * EMB_DIM))


def _tc_stage(z_e, zn, wt, wn):
    nblocks = BATCH // ROWS_PER_BLOCK
    return pl.pallas_call(
        _tc_body,
        grid=(nblocks,),
        in_specs=[
            pl.BlockSpec((ROWS_PER_BLOCK, EMB_DIM), lambda i: (i, 0)),
            pl.BlockSpec((ROWS_PER_BLOCK, NUM_HEADS), lambda i: (i, 0)),
            pl.BlockSpec((NUM_HEADS, HEAD_DIM, NUM_CODES), lambda i: (0, 0, 0)),
            pl.BlockSpec((NUM_HEADS, NUM_CODES), lambda i: (0, 0)),
        ],
        out_specs=[
            pl.BlockSpec((ROWS_PER_BLOCK, NUM_HEADS), lambda i: (i, 0)),
            pl.BlockSpec((ROWS_PER_BLOCK, NUM_HEADS), lambda i: (i, 0)),
            pl.BlockSpec(memory_space=pltpu.SMEM),
            pl.BlockSpec(memory_space=pltpu.SMEM),
        ],
        out_shape=[
            jax.ShapeDtypeStruct((BATCH, NUM_HEADS), jnp.int32),
            jax.ShapeDtypeStruct((BATCH, NUM_HEADS), jnp.int32),
            jax.ShapeDtypeStruct((1,), jnp.float32),
            jax.ShapeDtypeStruct((1,), jnp.float32),
        ],
        scratch_shapes=[
            pltpu.VMEM((NUM_HEADS, NUM_CODES), jnp.float32),
            pltpu.SMEM((1,), jnp.float32),
        ],
        compiler_params=pltpu.CompilerParams(
            dimension_semantics=("arbitrary",),
        ),
    )(z_e, zn, wt, wn)


def _sc_gather_body(wflat_hbm, gidx_hbm, out_hbm, idx_v, rows_v, sem):
    wid = lax.axis_index("s") * SC_CORES + lax.axis_index("c")
    base = wid * ROWS_PER_WORKER

    def body(i, carry):
        off = base + i * SC_CHUNK
        pltpu.sync_copy(gidx_hbm.at[pl.ds(off, SC_CHUNK)], idx_v)
        pltpu.async_copy(wflat_hbm.at[idx_v], rows_v, sem).wait()
        pltpu.sync_copy(rows_v, out_hbm.at[pl.ds(off, SC_CHUNK)])
        return carry

    lax.fori_loop(0, ROWS_PER_WORKER // SC_CHUNK, body, 0)


_sc_gather = functools.partial(
    pl.kernel,
    _sc_gather_body,
    out_type=jax.ShapeDtypeStruct((TOTAL_ROWS, HEAD_DIM), jnp.float32),
    mesh=plsc.VectorSubcoreMesh(core_axis_name="c", subcore_axis_name="s"),
    scratch_types=[
        pltpu.VMEM((SC_CHUNK,), jnp.int32),
        pltpu.VMEM((SC_CHUNK, HEAD_DIM), jnp.float32),
        pltpu.SemaphoreType.DMA,
    ],
)()


def kernel(z_e, emb_weights):
    zs = z_e.reshape(BATCH, NUM_HEADS, HEAD_DIM)
    zn = jnp.sum(zs * zs, axis=2)
    wt = jnp.transpose(emb_weights, (0, 2, 1))
    wn = jnp.sum(emb_weights * emb_weights, axis=2)

    idx, gidx, loss1, perp1 = _tc_stage(z_e, zn, wt, wn)

    wflat = emb_weights.reshape(NUM_HEADS * NUM_CODES, HEAD_DIM)
    zq_rows = _sc_gather(wflat, gidx.reshape(TOTAL_ROWS))
    z_q = zq_rows.reshape(BATCH, EMB_DIM)

    codebook_loss = loss1[0]
    commitment_loss = jnp.float32(COMMITMENT_COST) * codebook_loss
    perplexity = perp1[0]
    return (z_q, idx, codebook_loss, commitment_loss, perplexity)


# speedup vs baseline: 1.6916x; 1.6916x over previous
import functools

import jax
import jax.numpy as jnp
from jax import lax
from jax.experimental import pallas as pl
from jax.experimental.pallas import tpu as pltpu
from jax.experimental.pallas import tpu_sc as plsc

NUM_CODES = 1024
EMB_DIM = 256
NUM_HEADS = 4
HEAD_DIM = EMB_DIM // NUM_HEADS
COMMITMENT_COST = 0.1

BATCH = 65536
ROWS_PER_BLOCK = 512

SC_CORES = 2
SC_SUBCORES = 16
SC_WORKERS = SC_CORES * SC_SUBCORES
TOTAL_ROWS = BATCH * NUM_HEADS
ROWS_PER_WORKER = TOTAL_ROWS // SC_WORKERS
SC_CHUNK = 512


def _tc_body(z_ref, zn_ref, wt_ref, wn_ref,
             idx_ref, gidx_ref, loss_ref, perp_ref,
             counts, loss_acc):
    step = pl.program_id(0)
    nsteps = pl.num_programs(0)

    @pl.when(step == 0)
    def _init():
        counts[...] = jnp.zeros_like(counts)
        loss_acc[0] = jnp.float32(0.0)

    idx_cols = []
    total = jnp.float32(0.0)
    for h in range(NUM_HEADS):
        zh = z_ref[:, h * HEAD_DIM:(h + 1) * HEAD_DIM]
        w = wt_ref[h]
        mm = jnp.dot(zh, w, preferred_element_type=jnp.float32)
        zn = zn_ref[:, h:h + 1]
        wn = wn_ref[h:h + 1, :]
        sq = (zn + wn) - 2.0 * mm
        dist = jnp.sqrt(jnp.maximum(sq, 0.0))
        m = jnp.min(dist, axis=1, keepdims=True)
        iota = lax.broadcasted_iota(jnp.int32, dist.shape, 1)
        idxh = jnp.min(jnp.where(dist == m, iota, jnp.int32(NUM_CODES)), axis=1)
        idx_cols.append(idxh)
        onehot = (iota == idxh[:, None]).astype(jnp.float32)
        counts[h:h + 1, :] = counts[h:h + 1, :] + jnp.sum(
            onehot, axis=0, keepdims=True)
        total = total + jnp.sum(m[:, 0] * m[:, 0])

    idx_blk = jnp.stack(idx_cols, axis=1)
    idx_ref[...] = idx_blk
    offs = jnp.arange(NUM_HEADS, dtype=jnp.int32) * NUM_CODES
    gidx_ref[...] = idx_blk + offs[None, :]
    loss_acc[0] = loss_acc[0] + total

    @pl.when(step == nsteps - 1)
    def _fin():
        avg = counts[...] * jnp.float32(1.0 / BATCH)
        ent = -jnp.sum(avg * jnp.log(avg + 1e-10), axis=1, keepdims=True)
        perp_ref[0] = jnp.mean(jnp.exp(ent))
        loss_ref[0] = loss_acc[0] * jnp.float32(1.0 / (BATCH * EMB_DIM))


def _tc_stage(z_e, zn, wt, wn):
    nblocks = BATCH // ROWS_PER_BLOCK
    return pl.pallas_call(
        _tc_body,
        grid=(nblocks,),
        in_specs=[
            pl.BlockSpec((ROWS_PER_BLOCK, EMB_DIM), lambda i: (i, 0)),
            pl.BlockSpec((ROWS_PER_BLOCK, NUM_HEADS), lambda i: (i, 0)),
            pl.BlockSpec((NUM_HEADS, HEAD_DIM, NUM_CODES), lambda i: (0, 0, 0)),
            pl.BlockSpec((NUM_HEADS, NUM_CODES), lambda i: (0, 0)),
        ],
        out_specs=[
            pl.BlockSpec((ROWS_PER_BLOCK, NUM_HEADS), lambda i: (i, 0)),
            pl.BlockSpec((ROWS_PER_BLOCK, NUM_HEADS), lambda i: (i, 0)),
            pl.BlockSpec(memory_space=pltpu.SMEM),
            pl.BlockSpec(memory_space=pltpu.SMEM),
        ],
        out_shape=[
            jax.ShapeDtypeStruct((BATCH, NUM_HEADS), jnp.int32),
            jax.ShapeDtypeStruct((BATCH, NUM_HEADS), jnp.int32),
            jax.ShapeDtypeStruct((1,), jnp.float32),
            jax.ShapeDtypeStruct((1,), jnp.float32),
        ],
        scratch_shapes=[
            pltpu.VMEM((NUM_HEADS, NUM_CODES), jnp.float32),
            pltpu.SMEM((1,), jnp.float32),
        ],
        compiler_params=pltpu.CompilerParams(
            dimension_semantics=("arbitrary",),
        ),
    )(z_e, zn, wt, wn)


def _sc_gather_body(wflat_hbm, gidx_hbm, out_hbm, idx_v, rows_v, sem):
    wid = lax.axis_index("s") * SC_CORES + lax.axis_index("c")
    base = wid * ROWS_PER_WORKER

    def body(i, carry):
        off = base + i * SC_CHUNK
        pltpu.sync_copy(gidx_hbm.at[pl.ds(off, SC_CHUNK)], idx_v)
        pltpu.async_copy(wflat_hbm.at[idx_v], rows_v, sem).wait()
        pltpu.sync_copy(rows_v, out_hbm.at[pl.ds(off, SC_CHUNK)])
        return carry

    lax.fori_loop(0, ROWS_PER_WORKER // SC_CHUNK, body, 0)


def _sc_gather(wflat, gidx_flat):
    run = pl.kernel(
        _sc_gather_body,
        out_type=jax.ShapeDtypeStruct((TOTAL_ROWS, HEAD_DIM), jnp.float32),
        mesh=plsc.VectorSubcoreMesh(core_axis_name="c", subcore_axis_name="s"),
        scratch_types=[
            pltpu.VMEM((SC_CHUNK,), jnp.int32),
            pltpu.VMEM((SC_CHUNK, HEAD_DIM), jnp.float32),
            pltpu.SemaphoreType.DMA,
        ],
        compiler_params=pltpu.CompilerParams(use_tc_tiling_on_sc=False),
    )
    return run(wflat, gidx_flat)


def kernel(z_e, emb_weights):
    zs = z_e.reshape(BATCH, NUM_HEADS, HEAD_DIM)
    zn = jnp.sum(zs * zs, axis=2)
    wt = jnp.transpose(emb_weights, (0, 2, 1))
    wn = jnp.sum(emb_weights * emb_weights, axis=2)

    idx, gidx, loss1, perp1 = _tc_stage(z_e, zn, wt, wn)

    wflat = emb_weights.reshape(NUM_HEADS * NUM_CODES, HEAD_DIM)
    zq_rows = _sc_gather(wflat, gidx.reshape(TOTAL_ROWS))
    z_q = zq_rows.reshape(BATCH, EMB_DIM)

    codebook_loss = loss1[0]
    commitment_loss = jnp.float32(COMMITMENT_COST) * codebook_loss
    perplexity = perp1[0]
    return (z_q, idx, codebook_loss, commitment_loss, perplexity)


# speedup vs baseline: 1.8711x; 1.1061x over previous
import jax
import jax.numpy as jnp
from jax import lax
from jax.experimental import pallas as pl
from jax.experimental.pallas import tpu as pltpu
from jax.experimental.pallas import tpu_sc as plsc

NUM_CODES = 1024
EMB_DIM = 256
NUM_HEADS = 4
HEAD_DIM = EMB_DIM // NUM_HEADS
COMMITMENT_COST = 0.1

BATCH = 65536
ROWS_PER_BLOCK = 512

SC_CORES = 2
SC_SUBCORES = 16
SC_WORKERS = SC_CORES * SC_SUBCORES
TOTAL_ROWS = BATCH * NUM_HEADS
ROWS_PER_WORKER = TOTAL_ROWS // SC_WORKERS
SC_CHUNK = 1024
HIST_W = 16
NUM_GIDS = NUM_HEADS * NUM_CODES


def _tc_body(z_ref, zn_ref, wt2_ref, wn_ref,
             idx_ref, gidx_ref, loss_ref, loss_acc):
    step = pl.program_id(0)
    nsteps = pl.num_programs(0)

    @pl.when(step == 0)
    def _init():
        loss_acc[0] = jnp.float32(0.0)

    idx_cols = []
    total = jnp.float32(0.0)
    for h in range(NUM_HEADS):
        zh = z_ref[:, h * HEAD_DIM:(h + 1) * HEAD_DIM]
        w2 = wt2_ref[h]
        mm2 = jnp.dot(zh, w2, preferred_element_type=jnp.float32)
        zn = zn_ref[:, h:h + 1]
        wn = wn_ref[h:h + 1, :]
        sq = (zn + wn) + mm2
        dist = jnp.sqrt(jnp.maximum(sq, 0.0))
        m = jnp.min(dist, axis=1, keepdims=True)
        iota = lax.broadcasted_iota(jnp.int32, dist.shape, 1)
        idxh = jnp.min(jnp.where(dist == m, iota, jnp.int32(NUM_CODES)),
                       axis=1)
        idx_cols.append(idxh)
        total = total + jnp.sum(m[:, 0] * m[:, 0])

    idx_blk = jnp.stack(idx_cols, axis=1)
    idx_ref[...] = idx_blk
    offs = jnp.arange(NUM_HEADS, dtype=jnp.int32) * NUM_CODES
    gidx_ref[...] = idx_blk + offs[None, :]
    loss_acc[0] = loss_acc[0] + total

    @pl.when(step == nsteps - 1)
    def _fin():
        loss_ref[0] = loss_acc[0] * jnp.float32(1.0 / (BATCH * EMB_DIM))


def _tc_stage(z_e, zn, wt2, wn):
    nblocks = BATCH // ROWS_PER_BLOCK
    return pl.pallas_call(
        _tc_body,
        grid=(nblocks,),
        in_specs=[
            pl.BlockSpec((ROWS_PER_BLOCK, EMB_DIM), lambda i: (i, 0)),
            pl.BlockSpec((ROWS_PER_BLOCK, NUM_HEADS), lambda i: (i, 0)),
            pl.BlockSpec((NUM_HEADS, HEAD_DIM, NUM_CODES), lambda i: (0, 0, 0)),
            pl.BlockSpec((NUM_HEADS, NUM_CODES), lambda i: (0, 0)),
        ],
        out_specs=[
            pl.BlockSpec((ROWS_PER_BLOCK, NUM_HEADS), lambda i: (i, 0)),
            pl.BlockSpec((ROWS_PER_BLOCK, NUM_HEADS), lambda i: (i, 0)),
            pl.BlockSpec(memory_space=pltpu.SMEM),
        ],
        out_shape=[
            jax.ShapeDtypeStruct((BATCH, NUM_HEADS), jnp.int32),
            jax.ShapeDtypeStruct((BATCH, NUM_HEADS), jnp.int32),
            jax.ShapeDtypeStruct((1,), jnp.float32),
        ],
        scratch_shapes=[
            pltpu.SMEM((1,), jnp.float32),
        ],
        compiler_params=pltpu.CompilerParams(
            dimension_semantics=("arbitrary",),
        ),
    )(z_e, zn, wt2, wn)


def _sc_body(wflat_hbm, gidx_hbm, zeros_hbm, ones_hbm,
             out_hbm, hist_hbm,
             idx_v, rows_v, ones_v, hist_sh, sem):
    cid = lax.axis_index("c")
    sid = lax.axis_index("s")
    wid = sid * SC_CORES + cid
    base = wid * ROWS_PER_WORKER

    pltpu.sync_copy(ones_hbm, ones_v)

    @pl.when(sid == 0)
    def _init_hist():
        pltpu.sync_copy(zeros_hbm, hist_sh)

    plsc.subcore_barrier()

    def body(i, carry):
        off = base + i * SC_CHUNK
        pltpu.sync_copy(gidx_hbm.at[pl.ds(off, SC_CHUNK)], idx_v)
        pltpu.async_copy(wflat_hbm.at[idx_v], rows_v, sem).wait()
        pltpu.sync_copy(rows_v, out_hbm.at[pl.ds(off, SC_CHUNK)])
        pltpu.sync_copy(ones_v, hist_sh.at[idx_v], add=True)
        return carry

    lax.fori_loop(0, ROWS_PER_WORKER // SC_CHUNK, body, 0)

    plsc.subcore_barrier()

    @pl.when(sid == 0)
    def _emit_hist():
        pltpu.sync_copy(hist_sh, hist_hbm.at[cid])


def _sc_stage(wflat, gidx_flat, zeros_h, ones_h):
    run = pl.kernel(
        _sc_body,
        out_type=[
            jax.ShapeDtypeStruct((TOTAL_ROWS, HEAD_DIM), jnp.float32),
            jax.ShapeDtypeStruct((SC_CORES, NUM_GIDS, HIST_W), jnp.float32),
        ],
        mesh=plsc.VectorSubcoreMesh(core_axis_name="c", subcore_axis_name="s"),
        scratch_types=[
            pltpu.VMEM((SC_CHUNK,), jnp.int32),
            pltpu.VMEM((SC_CHUNK, HEAD_DIM), jnp.float32),
            pltpu.VMEM((SC_CHUNK, HIST_W), jnp.float32),
            pltpu.VMEM_SHARED((NUM_GIDS, HIST_W), jnp.float32),
            pltpu.SemaphoreType.DMA,
        ],
        compiler_params=pltpu.CompilerParams(use_tc_tiling_on_sc=False),
    )
    return run(wflat, gidx_flat, zeros_h, ones_h)


def _perp_body(hist_ref, perp_ref):
    h = hist_ref[...]
    c = h[0] + h[1]
    s = jnp.sum(c, axis=2)
    avg = s * jnp.float32(1.0 / (HIST_W * BATCH))
    ent = -jnp.sum(avg * jnp.log(avg + 1e-10), axis=1)
    perp_ref[0] = jnp.mean(jnp.exp(ent))


def _perp_stage(hist4):
    return pl.pallas_call(
        _perp_body,
        grid=(1,),
        in_specs=[
            pl.BlockSpec((SC_CORES, NUM_HEADS, NUM_CODES, HIST_W),
                         lambda i: (0, 0, 0, 0)),
        ],
        out_specs=pl.BlockSpec(memory_space=pltpu.SMEM),
        out_shape=jax.ShapeDtypeStruct((1,), jnp.float32),
    )(hist4)


def kernel(z_e, emb_weights):
    zs = z_e.reshape(BATCH, NUM_HEADS, HEAD_DIM)
    zn = jnp.sum(zs * zs, axis=2)
    wt2 = jnp.transpose(emb_weights, (0, 2, 1)) * jnp.float32(-2.0)
    wn = jnp.sum(emb_weights * emb_weights, axis=2)

    idx, gidx, loss1 = _tc_stage(z_e, zn, wt2, wn)

    wflat = emb_weights.reshape(NUM_GIDS, HEAD_DIM)
    zeros_h = jnp.zeros((NUM_GIDS, HIST_W), jnp.float32)
    ones_h = jnp.ones((SC_CHUNK, HIST_W), jnp.float32)
    zq_rows, hist = _sc_stage(wflat, gidx.reshape(TOTAL_ROWS), zeros_h, ones_h)
    z_q = zq_rows.reshape(BATCH, EMB_DIM)

    perplexity = _perp_stage(
        hist.reshape(SC_CORES, NUM_HEADS, NUM_CODES, HIST_W))[0]

    codebook_loss = loss1[0]
    commitment_loss = jnp.float32(COMMITMENT_COST) * codebook_loss
    return (z_q, idx, codebook_loss, commitment_loss, perplexity)
